# baseline (speedup 1.0000x reference)
import functools

import jax
import jax.numpy as jnp
from jax import lax
from jax.experimental import pallas as pl
from jax.experimental.pallas import tpu as pltpu


def _nnconv_kernel(ea0_ref, ea1_ref, ea2_ref, ea3_ref, na_ref, w1_ref, b1_ref, w2_ref, b2_ref,
                   root_ref, bias_ref, out_ref, hsum_ref, *, N, HID, IN_C,
                   OUT_C, CH, NC):
    c = pl.program_id(1)
    part = None
    for ref in (ea0_ref, ea1_ref, ea2_ref, ea3_ref):
        x = ref[0]
        h = jnp.maximum(
            jnp.dot(x, w1_ref[...], preferred_element_type=jnp.float32)
            + b1_ref[0], 0.0)
        p = jnp.sum(h.reshape(CH // N, N, HID), axis=0)
        part = p if part is None else part + p

    @pl.when(c == 0)
    def _():
        hsum_ref[...] = part

    @pl.when(c > 0)
    def _():
        hsum_ref[...] = hsum_ref[...] + part

    @pl.when(c == NC - 1)
    def _():
        ws = jnp.dot(hsum_ref[...], w2_ref[...],
                     preferred_element_type=jnp.float32) + N * b2_ref[0]
        na = na_ref[0]
        KO = IN_C * OUT_C
        R = (lax.broadcasted_iota(jnp.int32, (IN_C, KO), 1) // OUT_C ==
             lax.broadcasted_iota(jnp.int32, (IN_C, KO), 0)).astype(jnp.float32)
        S = (lax.broadcasted_iota(jnp.int32, (KO, OUT_C), 0) % OUT_C ==
             lax.broadcasted_iota(jnp.int32, (KO, OUT_C), 1)).astype(jnp.float32)
        msg = jnp.dot(jnp.dot(na, R, preferred_element_type=jnp.float32) * ws, S,
                      preferred_element_type=jnp.float32)
        out_ref[0] = msg + jnp.dot(na, root_ref[...],
                                   preferred_element_type=jnp.float32) + bias_ref[0]


def kernel(node_attr, edge_adj, W1, b1, W2, b2, root, bias):
    B, N, IN_C = node_attr.shape
    D_EDGE = edge_adj.shape[-1]
    HID = W1.shape[1]
    OUT_C = root.shape[1]
    NN = N * N
    CH = 4096
    NC = NN // (4 * CH)

    ea2 = edge_adj.reshape(B, NN, D_EDGE)
    b1r = b1.reshape(1, HID)
    b2r = b2.reshape(1, IN_C * OUT_C)
    biasr = bias.reshape(1, OUT_C)

    kern = functools.partial(_nnconv_kernel, N=N, HID=HID, IN_C=IN_C,
                             OUT_C=OUT_C, CH=CH, NC=NC)
    out = pl.pallas_call(
        kern,
        grid=(B, NC),
        in_specs=[
            pl.BlockSpec((1, CH, D_EDGE), lambda b, c: (b, 4 * c, 0)),
            pl.BlockSpec((1, CH, D_EDGE), lambda b, c: (b, 4 * c + 1, 0)),
            pl.BlockSpec((1, CH, D_EDGE), lambda b, c: (b, 4 * c + 2, 0)),
            pl.BlockSpec((1, CH, D_EDGE), lambda b, c: (b, 4 * c + 3, 0)),
            pl.BlockSpec((1, N, IN_C), lambda b, c: (b, 0, 0)),
            pl.BlockSpec((D_EDGE, HID), lambda b, c: (0, 0)),
            pl.BlockSpec((1, HID), lambda b, c: (0, 0)),
            pl.BlockSpec((HID, IN_C * OUT_C), lambda b, c: (0, 0)),
            pl.BlockSpec((1, IN_C * OUT_C), lambda b, c: (0, 0)),
            pl.BlockSpec((IN_C, OUT_C), lambda b, c: (0, 0)),
            pl.BlockSpec((1, OUT_C), lambda b, c: (0, 0)),
        ],
        out_specs=pl.BlockSpec((1, N, OUT_C), lambda b, c: (b, 0, 0)),
        out_shape=jax.ShapeDtypeStruct((B, N, OUT_C), jnp.float32),
        scratch_shapes=[pltpu.VMEM((N, HID), jnp.float32)],
        compiler_params=pltpu.CompilerParams(
            dimension_semantics=("parallel", "arbitrary")),
    )(ea2, ea2, ea2, ea2, node_attr, W1, b1r, W2, b2r, root, biasr)
    return out
